# Initial kernel scaffold; baseline (speedup 1.0000x reference)
import jax
import jax.numpy as jnp
from jax.experimental import pallas as pl
from jax.experimental.pallas import tpu as pltpu
from jax.experimental.pallas import tpu_sc as plsc

_BATCH = 16384
_HIST = 50
_DIM = 32
_NUM_IDX = _BATCH * _HIST
_WINDOW = 128


def kernel(x, table):
    idx = x.reshape(1, _NUM_IDX).astype(jnp.int32)
    mesh = plsc.VectorSubcoreMesh(core_axis_name="core", subcore_axis_name="subcore")

    @pl.kernel(
        out_type=jax.ShapeDtypeStruct((_NUM_IDX, _DIM), table.dtype),
        mesh=mesh,
    )
    def gather_kernel(table_hbm, idx_hbm, out_hbm):
        def body(i_vmem, o_vmem):
            pltpu.sync_copy(table_hbm.at[i_vmem.at[0]], o_vmem)

        pltpu.emit_pipeline(
            body,
            grid=(_NUM_IDX // _WINDOW,),
            in_specs=[pl.BlockSpec((1, _WINDOW), index_map=lambda i: (0, i))],
            out_specs=[pl.BlockSpec((_WINDOW, _DIM), index_map=lambda i: (i, 0))],
            core_axis_name=("core", "subcore"),
            dimension_semantics=(pltpu.PARALLEL,),
        )(idx_hbm, out_hbm)

    out = gather_kernel(table, idx)
    return out.reshape(_BATCH, _HIST, _DIM)


# speedup vs baseline: 1.0950x; 1.0950x over previous
import functools

import jax
import jax.numpy as jnp
from jax import lax
from jax.experimental import pallas as pl
from jax.experimental.pallas import tpu as pltpu
from jax.experimental.pallas import tpu_sc as plsc

_BATCH = 16384
_HIST = 50
_DIM = 32
_NUM_IDX = _BATCH * _HIST
_NUM_WORKERS = 32
_PER_WORKER = _NUM_IDX // _NUM_WORKERS
_CHUNK = 1024
_NUM_CHUNKS = _PER_WORKER // _CHUNK


def kernel(x, table):
    idx = x.reshape(_NUM_IDX).astype(jnp.int32)
    mesh = plsc.VectorSubcoreMesh(core_axis_name="c", subcore_axis_name="s")

    @functools.partial(
        pl.kernel,
        mesh=mesh,
        out_type=jax.ShapeDtypeStruct((_NUM_IDX, _DIM), jnp.float32),
        compiler_params=pltpu.CompilerParams(use_tc_tiling_on_sc=False),
        scratch_types=[
            pltpu.VMEM((_CHUNK,), jnp.int32),
            pltpu.VMEM((_CHUNK, _DIM), jnp.float32),
            pltpu.SemaphoreType.DMA,
        ],
    )
    def gather_kernel(table_hbm, idx_hbm, out_hbm, idx_v, rows_v, sem):
        wid = lax.axis_index("s") * 2 + lax.axis_index("c")
        base = wid * _PER_WORKER

        @pl.loop(0, _NUM_CHUNKS)
        def _(c):
            off = base + c * _CHUNK
            pltpu.sync_copy(idx_hbm.at[pl.ds(off, _CHUNK)], idx_v)
            pltpu.async_copy(table_hbm.at[idx_v], rows_v, sem).wait()
            pltpu.sync_copy(rows_v, out_hbm.at[pl.ds(off, _CHUNK)])

    out = gather_kernel(table, idx)
    return out.reshape(_BATCH, _HIST, _DIM)


# speedup vs baseline: 1.1131x; 1.0165x over previous
import functools

import jax
import jax.numpy as jnp
from jax import lax
from jax.experimental import pallas as pl
from jax.experimental.pallas import tpu as pltpu
from jax.experimental.pallas import tpu_sc as plsc

_BATCH = 16384
_HIST = 50
_DIM = 32
_NUM_IDX = _BATCH * _HIST
_NUM_WORKERS = 32
_PER_WORKER = _NUM_IDX // _NUM_WORKERS
_CHUNK = 1600
_NUM_CHUNKS = _PER_WORKER // _CHUNK


def kernel(x, table):
    idx = x.reshape(_NUM_IDX).astype(jnp.int32)
    mesh = plsc.VectorSubcoreMesh(core_axis_name="c", subcore_axis_name="s")

    @functools.partial(
        pl.kernel,
        mesh=mesh,
        out_type=jax.ShapeDtypeStruct((_NUM_IDX, _DIM), jnp.float32),
        compiler_params=pltpu.CompilerParams(use_tc_tiling_on_sc=False),
        scratch_types=[
            pltpu.VMEM((_CHUNK,), jnp.int32),
            pltpu.VMEM((_CHUNK,), jnp.int32),
            pltpu.VMEM((_CHUNK, _DIM), jnp.float32),
            pltpu.VMEM((_CHUNK, _DIM), jnp.float32),
            pltpu.SemaphoreType.DMA,
            pltpu.SemaphoreType.DMA,
            pltpu.SemaphoreType.DMA,
            pltpu.SemaphoreType.DMA,
        ],
    )
    def gather_kernel(
        table_hbm, idx_hbm, out_hbm, idx_v0, idx_v1, rows0, rows1, g0, g1, o0, o1
    ):
        wid = lax.axis_index("s") * 2 + lax.axis_index("c")
        base = wid * _PER_WORKER
        idx_v = (idx_v0, idx_v1)
        rows = (rows0, rows1)
        gsem = (g0, g1)
        osem = (o0, o1)

        def chunk_off(c):
            return base + c * _CHUNK

        gather_h = [None, None]
        out_h = [None, None]

        pltpu.sync_copy(idx_hbm.at[pl.ds(chunk_off(0), _CHUNK)], idx_v[0])
        gather_h[0] = pltpu.async_copy(table_hbm.at[idx_v[0]], rows[0], gsem[0])
        if _NUM_CHUNKS > 1:
            pltpu.sync_copy(idx_hbm.at[pl.ds(chunk_off(1), _CHUNK)], idx_v[1])

        for c in range(1, _NUM_CHUNKS):
            buf = c % 2
            prev = 1 - buf
            if out_h[buf] is not None:
                out_h[buf].wait()
            gather_h[buf] = pltpu.async_copy(
                table_hbm.at[idx_v[buf]], rows[buf], gsem[buf]
            )
            gather_h[prev].wait()
            out_h[prev] = pltpu.async_copy(
                rows[prev], out_hbm.at[pl.ds(chunk_off(c - 1), _CHUNK)], osem[prev]
            )
            if c + 1 < _NUM_CHUNKS:
                pltpu.sync_copy(
                    idx_hbm.at[pl.ds(chunk_off(c + 1), _CHUNK)], idx_v[prev]
                )

        last = _NUM_CHUNKS - 1
        buf = last % 2
        gather_h[buf].wait()
        out_h[buf] = pltpu.async_copy(
            rows[buf], out_hbm.at[pl.ds(chunk_off(last), _CHUNK)], osem[buf]
        )
        for h in out_h:
            if h is not None:
                h.wait()

    out = gather_kernel(table, idx)
    return out.reshape(_BATCH, _HIST, _DIM)
